# baseline (speedup 1.0000x reference)
import functools

import jax
import jax.numpy as jnp
from jax import lax
from jax.experimental import pallas as pl
from jax.experimental.pallas import tpu as pltpu
from jax.experimental.pallas import tpu_sc as plsc

GAMMA = 12.0
NC, NS, L = 2, 16, 16
NW = NC * NS
B, NNEG, D = 1024, 256, 128
DV = D // L
BPW = B // NW
RPW = BPW * NNEG
CHUNK = 128
NCHUNK = RPW // CHUNK
CPB = NNEG // CHUNK


def _sc_body(ent_hbm, rel_hbm, head_hbm, relidx_hbm, negidx_hbm, out_hbm,
             head_v, relidx_v, negidx_v, qe_v, qr_v, rows_v0, rows_v1, rows_v2, rows_v3, logits_v,
             sem0, sem1, sem2, sem3, semq, semq2):
    wid = lax.axis_index("s") * NC + lax.axis_index("c")

    pltpu.sync_copy(head_hbm.at[pl.ds(wid * BPW, BPW)], head_v)
    pltpu.sync_copy(relidx_hbm.at[pl.ds(wid * BPW, BPW)], relidx_v)
    pltpu.sync_copy(negidx_hbm.at[pl.ds(wid * NCHUNK, NCHUNK)], negidx_v)


    lane = lax.iota(jnp.int32, L)
    perms = [lax.bitwise_xor(lane, jnp.int32(1 << k)) for k in range(4)]
    dnums = lax.GatherDimensionNumbers(
        offset_dims=(), collapsed_slice_dims=(0,), start_index_map=(0,))

    def lanesum(x):
        for p in perms:
            x = x + lax.gather(x, p[:, None], dnums, slice_sizes=(1,),
                               mode=lax.GatherScatterMode.PROMISE_IN_BOUNDS)
        return x

    def start_gather(j, buf, sem):
        pltpu.async_copy(ent_hbm.at[negidx_v.at[j]], buf, sem)

    def wait_gather(j, buf, sem):
        pltpu.make_async_copy(ent_hbm.at[negidx_v.at[j]], buf, sem).wait()

    def compute_chunk(j, buf, qs):
        @pl.loop(0, CHUNK // L, unroll=2)
        def grp_loop(g):
            vec = jnp.zeros((L,), jnp.float32)
            for r in range(L):
                row = g * L + r
                acc = jnp.abs(buf[row, pl.ds(0, L)] - qs[0])
                for t in range(1, DV):
                    acc = acc + jnp.abs(buf[row, pl.ds(L * t, L)] - qs[t])
                vec = jnp.where(lane == r, GAMMA - lanesum(acc), vec)
            off = pl.multiple_of(j * CHUNK + g * L, L)
            logits_v[pl.ds(off, L)] = vec

    bufs = [rows_v0, rows_v1, rows_v2, rows_v3]
    sems = [sem0, sem1, sem2, sem3]
    NBUF = len(bufs)

    for k in range(NBUF):
        start_gather(k, bufs[k], sems[k])

    qe_d = pltpu.async_copy(ent_hbm.at[head_v], qe_v, semq)
    qr_d = pltpu.async_copy(rel_hbm.at[relidx_v], qr_v, semq2)
    qe_d.wait()
    qr_d.wait()

    @pl.loop(0, NCHUNK, step=NBUF)
    def chunk_loop(j):
        qs_pair = []
        for h in range(NBUF // CPB):
            lb = j // CPB + h
            qs_pair.append([qe_v[lb, pl.ds(L * t, L)] + qr_v[lb, pl.ds(L * t, L)]
                            for t in range(DV)])

        for k in range(NBUF):
            wait_gather(j + k, bufs[k], sems[k])
            compute_chunk(j + k, bufs[k], qs_pair[k // CPB])

            @pl.when(j + k + NBUF < NCHUNK)
            def _():
                start_gather(j + k + NBUF, bufs[k], sems[k])

    pltpu.sync_copy(logits_v, out_hbm.at[pl.ds(wid * RPW, RPW)])


@functools.partial(
    pl.kernel,
    out_type=jax.ShapeDtypeStruct((B * NNEG,), jnp.float32),
    mesh=plsc.VectorSubcoreMesh(core_axis_name="c", subcore_axis_name="s",
                                num_cores=NC, num_subcores=NS),
    scratch_types=[
        pltpu.VMEM((BPW,), jnp.int32),
        pltpu.VMEM((BPW,), jnp.int32),
        pltpu.VMEM((B * NNEG // NW // CHUNK, CHUNK), jnp.int32),
        pltpu.VMEM((BPW, D), jnp.float32),
        pltpu.VMEM((BPW, D), jnp.float32),
        pltpu.VMEM((CHUNK, D), jnp.float32),
        pltpu.VMEM((CHUNK, D), jnp.float32),
        pltpu.VMEM((CHUNK, D), jnp.float32),
        pltpu.VMEM((CHUNK, D), jnp.float32),
        pltpu.VMEM((RPW,), jnp.float32),
        pltpu.SemaphoreType.DMA,
        pltpu.SemaphoreType.DMA,
        pltpu.SemaphoreType.DMA,
        pltpu.SemaphoreType.DMA,
        pltpu.SemaphoreType.DMA,
        pltpu.SemaphoreType.DMA,
    ],
)
def _sc_kernel(ent_hbm, rel_hbm, head_hbm, relidx_hbm, negidx_hbm, out_hbm,
               head_v, relidx_v, negidx_v, qe_v, qr_v, rows_v0, rows_v1, rows_v2, rows_v3, logits_v,
               sem0, sem1, sem2, sem3, semq, semq2):
    _sc_body(ent_hbm, rel_hbm, head_hbm, relidx_hbm, negidx_hbm, out_hbm,
             head_v, relidx_v, negidx_v, qe_v, qr_v, rows_v0, rows_v1, rows_v2, rows_v3, logits_v,
             sem0, sem1, sem2, sem3, semq, semq2)


def kernel(entity_embedding, relation_embedding, head, relation, negative_items):
    negidx = negative_items.astype(jnp.int32).reshape(B * NNEG // CHUNK, CHUNK)
    out = _sc_kernel(entity_embedding, relation_embedding,
                     head.astype(jnp.int32), relation.astype(jnp.int32),
                     negidx)
    return out.reshape(B, NNEG)


# speedup vs baseline: 1.1691x; 1.1691x over previous
import functools

import jax
import jax.numpy as jnp
from jax import lax
from jax.experimental import pallas as pl
from jax.experimental.pallas import tpu as pltpu
from jax.experimental.pallas import tpu_sc as plsc

GAMMA = 12.0
NC, NS, L = 2, 16, 16
NW = NC * NS
B, NNEG, D = 1024, 256, 128
DV = D // L
BPW = B // NW
RPW = BPW * NNEG
CHUNK = 128
NCHUNK = RPW // CHUNK
CPB = NNEG // CHUNK


def _sc_body(ent_hbm, rel_hbm, head_hbm, relidx_hbm, negidx_hbm, out_hbm,
             head_v, relidx_v, negidx_v, qe_v, qr_v, rows_v0, rows_v1, rows_v2, rows_v3, logits_v,
             sem0, sem1, sem2, sem3, semq, semq2):
    wid = lax.axis_index("s") * NC + lax.axis_index("c")

    pltpu.sync_copy(head_hbm.at[pl.ds(wid * BPW, BPW)], head_v)
    pltpu.sync_copy(relidx_hbm.at[pl.ds(wid * BPW, BPW)], relidx_v)
    pltpu.sync_copy(negidx_hbm.at[pl.ds(wid * NCHUNK, NCHUNK)], negidx_v)


    lane = lax.iota(jnp.int32, L)
    perms = [lax.bitwise_xor(lane, jnp.int32(1 << k)) for k in range(4)]
    dnums = lax.GatherDimensionNumbers(
        offset_dims=(), collapsed_slice_dims=(0,), start_index_map=(0,))

    def lanesum(x):
        for p in perms:
            x = x + lax.gather(x, p[:, None], dnums, slice_sizes=(1,),
                               mode=lax.GatherScatterMode.PROMISE_IN_BOUNDS)
        return x

    def start_gather(j, buf, sem):
        pltpu.async_copy(ent_hbm.at[negidx_v.at[j]], buf, sem)

    def wait_gather(j, buf, sem):
        pltpu.make_async_copy(ent_hbm.at[negidx_v.at[j]], buf, sem).wait()

    def compute_chunk(j, buf, qs):
        @plsc.parallel_loop(0, CHUNK // L, unroll=2)
        def grp_loop(g):
            vec = jnp.zeros((L,), jnp.float32)
            for r in range(L):
                row = g * L + r
                acc = jnp.abs(buf[row, pl.ds(0, L)] - qs[0])
                for t in range(1, DV):
                    acc = acc + jnp.abs(buf[row, pl.ds(L * t, L)] - qs[t])
                vec = jnp.where(lane == r, GAMMA - lanesum(acc), vec)
            off = pl.multiple_of(j * CHUNK + g * L, L)
            logits_v[pl.ds(off, L)] = vec

    bufs = [rows_v0, rows_v1, rows_v2, rows_v3]
    sems = [sem0, sem1, sem2, sem3]
    NBUF = len(bufs)

    for k in range(NBUF):
        start_gather(k, bufs[k], sems[k])

    qe_d = pltpu.async_copy(ent_hbm.at[head_v], qe_v, semq)
    qr_d = pltpu.async_copy(rel_hbm.at[relidx_v], qr_v, semq2)
    qe_d.wait()
    qr_d.wait()

    @pl.loop(0, NCHUNK, step=NBUF)
    def chunk_loop(j):
        qs_pair = []
        for h in range(NBUF // CPB):
            lb = j // CPB + h
            qs_pair.append([qe_v[lb, pl.ds(L * t, L)] + qr_v[lb, pl.ds(L * t, L)]
                            for t in range(DV)])

        for k in range(NBUF):
            wait_gather(j + k, bufs[k], sems[k])
            compute_chunk(j + k, bufs[k], qs_pair[k // CPB])

            @pl.when(j + k + NBUF < NCHUNK)
            def _():
                start_gather(j + k + NBUF, bufs[k], sems[k])

    pltpu.sync_copy(logits_v, out_hbm.at[pl.ds(wid * RPW, RPW)])


@functools.partial(
    pl.kernel,
    out_type=jax.ShapeDtypeStruct((B * NNEG,), jnp.float32),
    mesh=plsc.VectorSubcoreMesh(core_axis_name="c", subcore_axis_name="s",
                                num_cores=NC, num_subcores=NS),
    scratch_types=[
        pltpu.VMEM((BPW,), jnp.int32),
        pltpu.VMEM((BPW,), jnp.int32),
        pltpu.VMEM((B * NNEG // NW // CHUNK, CHUNK), jnp.int32),
        pltpu.VMEM((BPW, D), jnp.float32),
        pltpu.VMEM((BPW, D), jnp.float32),
        pltpu.VMEM((CHUNK, D), jnp.float32),
        pltpu.VMEM((CHUNK, D), jnp.float32),
        pltpu.VMEM((CHUNK, D), jnp.float32),
        pltpu.VMEM((CHUNK, D), jnp.float32),
        pltpu.VMEM((RPW,), jnp.float32),
        pltpu.SemaphoreType.DMA,
        pltpu.SemaphoreType.DMA,
        pltpu.SemaphoreType.DMA,
        pltpu.SemaphoreType.DMA,
        pltpu.SemaphoreType.DMA,
        pltpu.SemaphoreType.DMA,
    ],
)
def _sc_kernel(ent_hbm, rel_hbm, head_hbm, relidx_hbm, negidx_hbm, out_hbm,
               head_v, relidx_v, negidx_v, qe_v, qr_v, rows_v0, rows_v1, rows_v2, rows_v3, logits_v,
               sem0, sem1, sem2, sem3, semq, semq2):
    _sc_body(ent_hbm, rel_hbm, head_hbm, relidx_hbm, negidx_hbm, out_hbm,
             head_v, relidx_v, negidx_v, qe_v, qr_v, rows_v0, rows_v1, rows_v2, rows_v3, logits_v,
             sem0, sem1, sem2, sem3, semq, semq2)


def kernel(entity_embedding, relation_embedding, head, relation, negative_items):
    negidx = negative_items.astype(jnp.int32).reshape(B * NNEG // CHUNK, CHUNK)
    out = _sc_kernel(entity_embedding, relation_embedding,
                     head.astype(jnp.int32), relation.astype(jnp.int32),
                     negidx)
    return out.reshape(B, NNEG)
